# Initial kernel scaffold; baseline (speedup 1.0000x reference)
import jax
import jax.numpy as jnp
from jax.experimental import pallas as pl


def kernel(x, edge_index, cycle_index, batch, W_emb, b_emb, conv0_W, conv0_as, conv0_ad, conv0_b, conv1_W, conv1_as, conv1_ad, conv1_b, conv2_W, conv2_as, conv2_ad, conv2_b, bn0_g, bn0_b, bn1_g, bn1_b, bn2_g, bn2_b, lin1_W, lin1_b, lin2_W, lin2_b, lin3_W, lin3_b):
    raise NotImplementedError("write your pallas kernel here")


# speedup vs baseline: 48.7463x; 48.7463x over previous
import functools

import jax
import jax.numpy as jnp
from jax import lax
from jax.experimental import pallas as pl
from jax.experimental.pallas import tpu as pltpu
from jax.experimental.pallas import tpu_sc as plsc

_N = 10000
_HID = 128
_G = 64
_NC = 2
_NS = 16
_K = 128
_NXL = 10008
_NTAB = 10240
_TAB_SLAB = _NTAB // _NS
_ACC_SLAB = _N // _NS
_ZCH = _ACC_SLAB // 5

_GATHER_DNUMS = lax.GatherDimensionNumbers(
    offset_dims=(), collapsed_slice_dims=(0,), start_index_map=(0,))


def _perm(vec, idx16):
    return lax.gather(vec, idx16.reshape(16, 1), _GATHER_DNUMS, (1,),
                      mode=lax.GatherScatterMode.PROMISE_IN_BOUNDS)


def _bcast_lane(vec, j):
    return _perm(vec, jnp.full((16,), j, jnp.int32))


def _rot8(vec):
    return _perm(vec, (lax.iota(jnp.int32, 16) % 8) + 8)


def _sc_gat(xlp, tab, mv2d, src, dst, heads):
    e_pad = src.shape[0]
    chunks = e_pad // (_NC * _NS * _K)
    mesh = plsc.VectorSubcoreMesh(core_axis_name="c", subcore_axis_name="s")

    def body(xlp_hbm, tab_hbm, m_hbm, src_hbm, dst_hbm, num_hbm, den_hbm,
             num_sh, den_sh, tab_sh, idx_s, idx_d, rows, gs_b, gd_b,
             e_b, t1d, t2d, m_b, sem_r, sem_a, sem_d):
        c = lax.axis_index("c")
        s = lax.axis_index("s")
        zero = jnp.zeros((16,), jnp.float32)
        pltpu.sync_copy(m_hbm, m_b)

        @pl.loop(0, _K)
        def _zero_bufs(k):
            for j in range(8):
                rows[k, pl.ds(16 * j, 16)] = zero
            e_b[k, :] = zero

        acc_base = s * _ACC_SLAB
        for t in range(5):
            pltpu.sync_copy(rows.at[pl.ds(0, _ZCH)],
                            num_sh.at[pl.ds(acc_base + t * _ZCH, _ZCH)])
            pltpu.sync_copy(e_b.at[pl.ds(0, _ZCH)],
                            den_sh.at[pl.ds(acc_base + t * _ZCH, _ZCH)])
        tab_base = s * _TAB_SLAB
        for t in range(_TAB_SLAB // _K):
            pltpu.sync_copy(
                tab_hbm.at[pl.ds((tab_base + t * _K) * 16, _K * 16)], t1d)

            @pl.loop(0, _K)
            def _to2d(r):
                t2d[r, :] = t1d[pl.ds(r * 16, 16)]

            pltpu.sync_copy(t2d, tab_sh.at[pl.ds(tab_base + t * _K, _K)])
        plsc.subcore_barrier()

        tile_edges = chunks * _K
        tile_base = (c * _NS + s) * tile_edges
        mv = m_b[:]

        @pl.loop(0, chunks)
        def _chunk(g):
            off = tile_base + g * _K
            pltpu.sync_copy(src_hbm.at[pl.ds(off, _K)], idx_s)
            pltpu.sync_copy(dst_hbm.at[pl.ds(off, _K)], idx_d)
            cp_r = pltpu.async_copy(xlp_hbm.at[idx_s], rows, sem_r)
            cp_a = pltpu.async_copy(tab_sh.at[idx_s], gs_b, sem_a)
            cp_d = pltpu.async_copy(tab_sh.at[idx_d], gd_b, sem_d)
            cp_a.wait()
            cp_d.wait()
            cp_r.wait()

            @pl.loop(0, _K)
            def _edge(k):
                a = gs_b[k, :] + _rot8(gd_b[k, :])
                a = jnp.where(a > 0.0, a, 0.2 * a)
                ev = jnp.exp(a - mv)
                e_b[k, :] = ev
                if heads == 1:
                    ej = _bcast_lane(ev, 0)
                    for j in range(8):
                        rows[k, pl.ds(16 * j, 16)] = (
                            rows[k, pl.ds(16 * j, 16)] * ej)
                else:
                    for j in range(8):
                        ej = _bcast_lane(ev, j)
                        rows[k, pl.ds(16 * j, 16)] = (
                            rows[k, pl.ds(16 * j, 16)] * ej)

            pltpu.sync_copy(rows, num_sh.at[idx_d], add=True)
            pltpu.sync_copy(e_b, den_sh.at[idx_d], add=True)

        plsc.subcore_barrier()
        pltpu.sync_copy(num_sh.at[pl.ds(acc_base, _ACC_SLAB)],
                        num_hbm.at[c, pl.ds(acc_base, _ACC_SLAB)])
        for t in range(5):
            pltpu.sync_copy(den_sh.at[pl.ds(acc_base + t * _ZCH, _ZCH)],
                            t2d.at[pl.ds(0, _ZCH)])

            @pl.loop(0, _ZCH)
            def _to1d(r):
                t1d[pl.ds(r * 16, 16)] = t2d[r, :]

            pltpu.sync_copy(
                t1d.at[pl.ds(0, _ZCH * 16)],
                den_hbm.at[pl.ds((c * _N + acc_base + t * _ZCH) * 16,
                                 _ZCH * 16)])

    f = pl.kernel(
        body,
        out_type=[
            jax.ShapeDtypeStruct((_NC, _N, _HID), jnp.float32),
            jax.ShapeDtypeStruct((_NC * _N * 16,), jnp.float32),
        ],
        mesh=mesh,
        scratch_types=[
            pltpu.VMEM_SHARED((_N, _HID), jnp.float32),
            pltpu.VMEM_SHARED((_N, 16), jnp.float32),
            pltpu.VMEM_SHARED((_NTAB, 16), jnp.float32),
            pltpu.VMEM((_K,), jnp.int32),
            pltpu.VMEM((_K,), jnp.int32),
            pltpu.VMEM((_K, _HID), jnp.float32),
            pltpu.VMEM((_K, 16), jnp.float32),
            pltpu.VMEM((_K, 16), jnp.float32),
            pltpu.VMEM((_K, 16), jnp.float32),
            pltpu.VMEM((_K * 16,), jnp.float32),
            pltpu.VMEM((_K, 16), jnp.float32),
            pltpu.VMEM((16,), jnp.float32),
            pltpu.SemaphoreType.DMA,
            pltpu.SemaphoreType.DMA,
            pltpu.SemaphoreType.DMA,
        ],
        compiler_params=pltpu.CompilerParams(use_tc_tiling_on_sc=False),
    )
    num, den_flat = f(xlp, tab.reshape(-1), mv2d.reshape(-1), src, dst)
    return num, den_flat.reshape(_NC, _N, 16)


def _prep_tail(xl, as_ref, ad_ref, xlp_ref, tab_ref, m_ref):
    als = jnp.dot(xl, as_ref[...], preferred_element_type=jnp.float32,
                  precision=lax.Precision.HIGHEST)
    ald = jnp.dot(xl, ad_ref[...], preferred_element_type=jnp.float32,
                  precision=lax.Precision.HIGHEST)
    m = (jnp.max(als, axis=0, keepdims=True)
         + jnp.max(ald, axis=0, keepdims=True))
    c = jnp.where(m > 0.0, m, 0.2 * m)
    m_ref[...] = jnp.concatenate([c, c], axis=1)
    xlp_ref[0:_N, :] = xl
    xlp_ref[_N:_NXL, :] = jnp.zeros((_NXL - _N, _HID), jnp.float32)
    tab_ref[0:_N, :] = jnp.concatenate([als, ald], axis=1)
    tab_ref[_N:_NTAB, :] = jnp.concatenate(
        [jnp.full((_NTAB - _N, 8), -1e30, jnp.float32),
         jnp.zeros((_NTAB - _N, 8), jnp.float32)], axis=1)


def _embed_body(x_ref, we_ref, be_ref, w0_ref, as_ref, ad_ref,
                h_ref, xlp_ref, tab_ref, m_ref):
    h = jnp.dot(x_ref[...], we_ref[...],
                preferred_element_type=jnp.float32,
                precision=lax.Precision.HIGHEST) + be_ref[...]
    h_ref[...] = h
    xl = jnp.dot(h, w0_ref[...], preferred_element_type=jnp.float32,
                precision=lax.Precision.HIGHEST)
    _prep_tail(xl, as_ref, ad_ref, xlp_ref, tab_ref, m_ref)


def _merge_core(heads, num_ref, den_ref, cb_ref, g_ref, b_ref, hp_ref):
    nm = num_ref[0, :, :] + num_ref[1, :, :]
    dn = den_ref[0, :, :] + den_ref[1, :, :]
    jj = lax.broadcasted_iota(jnp.int32, (16, _HID), 0)
    ff = lax.broadcasted_iota(jnp.int32, (16, _HID), 1)
    if heads == 8:
        expm = (ff // 16) == jj
    else:
        expm = jj == 0
    den_feat = jnp.dot(dn, expm.astype(jnp.float32),
                       preferred_element_type=jnp.float32,
                       precision=lax.Precision.HIGHEST)
    out = nm / (den_feat + 1e-16) + cb_ref[...]
    mu = jnp.mean(out, axis=0, keepdims=True)
    var = jnp.mean((out - mu) ** 2, axis=0, keepdims=True)
    bn = (out - mu) / jnp.sqrt(var + 1e-5) * g_ref[...] + b_ref[...]
    return jnp.maximum(bn, 0.0) + hp_ref[...]


def _merge_body(heads, num_ref, den_ref, cb_ref, g_ref, b_ref, hp_ref,
                wn_ref, asn_ref, adn_ref, hn_ref, xlp_ref, tab_ref, m_ref):
    hn = _merge_core(heads, num_ref, den_ref, cb_ref, g_ref, b_ref, hp_ref)
    hn_ref[...] = hn
    xl = jnp.dot(hn, wn_ref[...], preferred_element_type=jnp.float32,
                precision=lax.Precision.HIGHEST)
    _prep_tail(xl, asn_ref, adn_ref, xlp_ref, tab_ref, m_ref)


def _final_body(num_ref, den_ref, cb_ref, g_ref, b_ref, hp_ref, bt_ref,
                l1w_ref, l1b_ref, l2w_ref, l2b_ref, l3w_ref, l3b_ref, o_ref):
    h3 = _merge_core(1, num_ref, den_ref, cb_ref, g_ref, b_ref, hp_ref)
    bt = bt_ref[...]
    gi = lax.broadcasted_iota(jnp.int32, (_G, _N), 0)
    onehot = (gi == bt).astype(jnp.float32)
    sums = jnp.dot(onehot, h3, preferred_element_type=jnp.float32,
                   precision=lax.Precision.HIGHEST)
    cnt = jnp.sum(onehot, axis=1, keepdims=True)
    pooled = sums / jnp.maximum(cnt, 1.0)
    o = jnp.maximum(jnp.dot(pooled, l1w_ref[...],
                            preferred_element_type=jnp.float32,
                precision=lax.Precision.HIGHEST)
                    + l1b_ref[...], 0.0)
    o = jnp.maximum(jnp.dot(o, l2w_ref[...],
                            preferred_element_type=jnp.float32,
                precision=lax.Precision.HIGHEST)
                    + l2b_ref[...], 0.0)
    o_ref[...] = jnp.dot(o, l3w_ref[...],
                         preferred_element_type=jnp.float32,
                precision=lax.Precision.HIGHEST) + l3b_ref[...]


def _as8(a):
    h = jnp.arange(8)
    cp = jnp.arange(_HID)
    val = a[h[None, :], (cp % 16)[:, None]]
    mask = (cp[:, None] // 16) == h[None, :]
    return jnp.where(mask, val, 0.0).astype(jnp.float32)


def _as1(a):
    return jnp.broadcast_to(a.reshape(_HID, 1), (_HID, 8)).astype(jnp.float32)


_PREP_OUT = [
    jax.ShapeDtypeStruct((_N, _HID), jnp.float32),
    jax.ShapeDtypeStruct((_NXL, _HID), jnp.float32),
    jax.ShapeDtypeStruct((_NTAB, 16), jnp.float32),
    jax.ShapeDtypeStruct((1, 16), jnp.float32),
]


def kernel(x, edge_index, cycle_index, batch, W_emb, b_emb,
           conv0_W, conv0_as, conv0_ad, conv0_b,
           conv1_W, conv1_as, conv1_ad, conv1_b,
           conv2_W, conv2_as, conv2_ad, conv2_b,
           bn0_g, bn0_b, bn1_g, bn1_b, bn2_g, bn2_b,
           lin1_W, lin1_b, lin2_W, lin2_b, lin3_W, lin3_b):
    ei = edge_index.astype(jnp.int32)
    e = ei.shape[1]
    loop = jnp.arange(_N, dtype=jnp.int32)
    e_tot = e + _N
    chunk_sz = _NC * _NS * _K
    e_pad = ((e_tot + chunk_sz - 1) // chunk_sz) * chunk_sz
    pad = e_pad - e_tot
    src = jnp.concatenate([ei[0], loop, jnp.full((pad,), _N, jnp.int32)])
    dst = jnp.concatenate([ei[1], loop, jnp.zeros((pad,), jnp.int32)])

    as0, ad0 = _as8(conv0_as), _as8(conv0_ad)
    as1, ad1 = _as8(conv1_as), _as8(conv1_ad)
    as2, ad2 = _as1(conv2_as), _as1(conv2_ad)
    r1 = lambda v: v.reshape(1, -1)

    _tc_params = pltpu.CompilerParams(vmem_limit_bytes=67108864)
    h0, xlp0, tab0, mv0 = pl.pallas_call(
        _embed_body, out_shape=_PREP_OUT,
        compiler_params=_tc_params)(
        x, W_emb, r1(b_emb), conv0_W, as0, ad0)
    num0, den0 = _sc_gat(xlp0, tab0, mv0, src, dst, heads=8)

    h1, xlp1, tab1, mv1 = pl.pallas_call(
        functools.partial(_merge_body, 8), out_shape=_PREP_OUT,
        compiler_params=_tc_params)(
        num0, den0, r1(conv0_b), r1(bn0_g), r1(bn0_b), h0,
        conv1_W, as1, ad1)
    num1, den1 = _sc_gat(xlp1, tab1, mv1, src, dst, heads=8)

    h2, xlp2, tab2, mv2 = pl.pallas_call(
        functools.partial(_merge_body, 8), out_shape=_PREP_OUT,
        compiler_params=_tc_params)(
        num1, den1, r1(conv1_b), r1(bn1_g), r1(bn1_b), h1,
        conv2_W, as2, ad2)
    num2, den2 = _sc_gat(xlp2, tab2, mv2, src, dst, heads=1)

    bt = batch.astype(jnp.int32).reshape(1, _N)
    o = pl.pallas_call(
        _final_body,
        out_shape=jax.ShapeDtypeStruct((_G, 128), jnp.float32),
        compiler_params=_tc_params)(
        num2, den2, r1(conv2_b), r1(bn2_g), r1(bn2_b), h2, bt,
        lin1_W, r1(lin1_b), lin2_W, r1(lin2_b), lin3_W, r1(lin3_b))
    return o
